# baseline (speedup 1.0000x reference)
import functools

import jax
import jax.numpy as jnp
from jax.experimental import pallas as pl
from jax.experimental.pallas import tpu as pltpu

_TAU1 = 10.0
_TM = 640
_TN = 1280


def _act(v):
    return jnp.clip(v, 0.0, 1.0)


def _split_dot(a, b_bf16):
    a_hi = a.astype(jnp.bfloat16)
    a_lo = (a - a_hi.astype(jnp.float32)).astype(jnp.bfloat16)
    return (jnp.dot(a_hi, b_bf16, preferred_element_type=jnp.float32)
            + jnp.dot(a_lo, b_bf16, preferred_element_type=jnp.float32))


def _split_dot_nt(a, bt_bf16):
    nt = (((1,), (1,)), ((), ()))
    a_hi = a.astype(jnp.bfloat16)
    a_lo = (a - a_hi.astype(jnp.float32)).astype(jnp.bfloat16)
    return (jax.lax.dot_general(a_hi, bt_bf16, nt,
                                preferred_element_type=jnp.float32)
            + jax.lax.dot_general(a_lo, bt_bf16, nt,
                                  preferred_element_type=jnp.float32))


def _ab_kernel(e2t_ref, t2et_ref, t2rt_ref, x_ref, wall_ref,
               e2tb_ref, t2etb_ref, e2r_ref, s0_ref, hx_ref, sr_ref):
    nt = (((1,), (1,)), ((), ()))
    e2t_b = e2t_ref[...].astype(jnp.bfloat16)
    t2et_b = t2et_ref[...].astype(jnp.bfloat16)
    e2tb_ref[...] = e2t_b
    t2etb_ref[...] = t2et_b
    t2rt = t2rt_ref[...]
    t2rt_b = t2rt.astype(jnp.bfloat16)
    sr = jnp.dot(wall_ref[...], t2rt,
                 preferred_element_type=jnp.float32)
    sr_ref[...] = sr

    @pl.when(pl.program_id(0) == 0)
    def _():
        e2r_ref[...] = jnp.zeros_like(e2r_ref)
        s0_ref[...] = jnp.zeros_like(s0_ref)
        hx_ref[...] = jnp.zeros_like(hx_ref)

    e2r_ref[...] += jax.lax.dot_general(t2rt_b, t2et_b, nt,
                                        preferred_element_type=jnp.float32)
    xo = _split_dot(x_ref[...], e2t_b)
    hx_ref[...] += jax.lax.dot_general(xo.astype(jnp.bfloat16), t2rt_b, nt,
                                       preferred_element_type=jnp.float32)
    z = jnp.concatenate([xo * sr[l:l + 1, :] for l in range(3)],
                        axis=0)
    s0_ref[...] += _split_dot_nt(z, t2et_b)


def _pass_ab(e2triple, t2e_t, t2r_t, input_x, w_all):
    n_ent, n_tr = e2triple.shape
    n_rel = t2r_t.shape[0]
    batch = input_x.shape[0]
    grid = (n_tr // _TM,)
    return pl.pallas_call(
        _ab_kernel,
        grid=grid,
        in_specs=[
            pl.BlockSpec((n_ent, _TM), lambda i: (0, i)),
            pl.BlockSpec((n_ent, _TM), lambda i: (0, i)),
            pl.BlockSpec((n_rel, _TM), lambda i: (0, i)),
            pl.BlockSpec((batch, n_ent), lambda i: (0, 0)),
            pl.BlockSpec((16, n_rel), lambda i: (0, 0)),
        ],
        out_specs=[
            pl.BlockSpec((n_ent, _TM), lambda i: (0, i)),
            pl.BlockSpec((n_ent, _TM), lambda i: (0, i)),
            pl.BlockSpec((n_rel, n_ent), lambda i: (0, 0)),
            pl.BlockSpec((3 * batch, n_ent), lambda i: (0, 0)),
            pl.BlockSpec((batch, n_rel), lambda i: (0, 0)),
            pl.BlockSpec((16, _TM), lambda i: (0, i)),
        ],
        out_shape=[
            jax.ShapeDtypeStruct((n_ent, n_tr), jnp.bfloat16),
            jax.ShapeDtypeStruct((n_ent, n_tr), jnp.bfloat16),
            jax.ShapeDtypeStruct((n_rel, n_ent), jnp.float32),
            jax.ShapeDtypeStruct((3 * batch, n_ent), jnp.float32),
            jax.ShapeDtypeStruct((batch, n_rel), jnp.float32),
            jax.ShapeDtypeStruct((16, n_tr), jnp.float32),
        ],
    )(e2triple, t2e_t, t2r_t, input_x, w_all)


def _cd_kernel(t_idx, last, nsteps,
               smu_ref, sdel_ref, e2t_ref, t2et_ref, sr_ref, hid_ref,
               out_ref, acc_ref):
    i = pl.program_id(0)

    @pl.when(i == 0)
    def _():
        acc_ref[...] = jnp.zeros_like(acc_ref)

    nt = (((1,), (1,)), ((), ()))
    e2t = e2t_ref[...]
    t2et = t2et_ref[...]
    mu = smu_ref[:, 0:1]
    ones8 = jnp.ones((8, e2t.shape[0]), jnp.bfloat16)
    colsum = jnp.dot(ones8, e2t, preferred_element_type=jnp.float32)[0:1, :]
    y = (jnp.dot(sdel_ref[...], e2t, preferred_element_type=jnp.float32)
         + mu * colsum)
    sr3 = sr_ref[3 * t_idx:3 * t_idx + 3, :]
    srt = _split_dot_nt(sr3, t2et)
    muy = jnp.mean(y, axis=1, keepdims=True)
    dz = jnp.concatenate(
        [(y[32 * l:32 * (l + 1), :] - muy[32 * l:32 * (l + 1), :])
         * sr3[l:l + 1, :] for l in range(3)],
        axis=0).astype(jnp.bfloat16)
    dacc = jax.lax.dot_general(dz, t2et, nt,
                               preferred_element_type=jnp.float32)
    rank1 = jnp.concatenate(
        [muy[32 * l:32 * (l + 1), :] * srt[l:l + 1, :] for l in range(3)],
        axis=0)
    acc_ref[...] += dacc + rank1

    @pl.when(i == nsteps - 1)
    def _():
        acc = acc_ref[...]
        if last:
            out_ref[...] = (acc[0:32, :] * hid_ref[0:1, :]
                            + acc[32:64, :] * hid_ref[1:2, :]
                            + acc[64:96, :] * hid_ref[2:3, :])
        else:
            out_ref[...] = jnp.concatenate(
                [acc[32 * l:32 * (l + 1), :] * hid_ref[l:l + 1, :]
                 for l in range(3)], axis=0)


def _pass_cd(t_idx, last, s, e2t_b, t2et_b, sr_all, hid):
    rows, n_ent = s.shape
    n_tr = e2t_b.shape[1]
    nsteps = n_tr // _TN
    mu = jnp.mean(s, axis=1, keepdims=True)
    s_mu = mu + jnp.zeros((rows, 128), jnp.float32)
    s_del = (s - mu).astype(jnp.bfloat16)
    out_shape = (jax.ShapeDtypeStruct((32, n_ent), jnp.float32) if last
                 else jax.ShapeDtypeStruct((rows, n_ent), jnp.float32))
    out_rows = 32 if last else rows
    return pl.pallas_call(
        functools.partial(_cd_kernel, t_idx, last, nsteps),
        grid=(nsteps,),
        in_specs=[
            pl.BlockSpec((rows, 128), lambda i: (0, 0)),
            pl.BlockSpec((rows, n_ent), lambda i: (0, 0)),
            pl.BlockSpec((n_ent, _TN), lambda i: (0, i)),
            pl.BlockSpec((n_ent, _TN), lambda i: (0, i)),
            pl.BlockSpec((16, _TN), lambda i: (0, i)),
            pl.BlockSpec((8, n_ent), lambda i: (0, 0)),
        ],
        out_specs=pl.BlockSpec((out_rows, n_ent), lambda i: (0, 0)),
        out_shape=out_shape,
        scratch_shapes=[pltpu.VMEM((rows, n_ent), jnp.float32)],
    )(s_mu, s_del, e2t_b, t2et_b, sr_all, hid)


def kernel(input_x, type, e2triple, triple2e, triple2r, w, weight, h, h_x,
           h_type, h_x_type, alpha, beta, alpha_x, beta_x, flag):
    type_m = type
    Tn, Ln, n = w.shape
    batch, n_ent = input_x.shape
    half = (n - 1) // 2
    flag_b = flag != 0

    w_probs_l, h_probs_l, h_type_probs_l, a_t_l, b_t_l = [], [], [], [], []
    for t in range(Tn):
        wp = jnp.where(flag_b, w[Tn - 1 - t], w[t])
        wp = jax.nn.softmax(wp, axis=-1)
        wp_flip = jnp.concatenate(
            [wp[:, half:-1], wp[:, :half], wp[:, -1:]], axis=-1)
        w_probs_l.append(jnp.where(flag_b, wp_flip, wp))
        if t == Tn - 1:
            h_probs_l.append(jnp.where(flag_b, h_x, h[t]))
            h_type_probs_l.append(jnp.where(flag_b, h_x_type, h_type[t]))
            a_t_l.append(jnp.where(flag_b, alpha_x, alpha[t]))
            b_t_l.append(jnp.where(flag_b, beta_x, beta[t]))
        else:
            h_probs_l.append(jnp.where(flag_b, h[Tn - 2 - t], h[t]))
            h_type_probs_l.append(jnp.where(flag_b, h_type[Tn - 2 - t],
                                            h_type[t]))
            a_t_l.append(jnp.where(flag_b, alpha[Tn - 2 - t], alpha[t]))
            b_t_l.append(jnp.where(flag_b, beta[Tn - 2 - t], beta[t]))
    h_x_probs = jnp.where(flag_b, h[-1], h_x)
    h_x_type_probs = jnp.where(flag_b, h_type[-1], h_x_type)
    a_x = jnp.where(flag_b, alpha[-1], alpha_x)
    b_x = jnp.where(flag_b, beta[-1], beta_x)

    w_stack = jnp.concatenate(w_probs_l, axis=0)
    w_all = jnp.zeros((16, n), jnp.float32).at[:3 * Tn].set(w_stack)

    e2t_b, t2et_b, e2r, s0pre, hx_acc, sr_all = _pass_ab(
        e2triple, triple2e.T, triple2r.T, input_x, w_all)

    hidden_e_t = _act(e2r)[:-1, :]
    hiddens = []
    for t in range(Tn):
        a_t_a = _act(a_t_l[t] / _TAU1)
        b_t_a = _act(b_t_l[t] / _TAU1)
        h_probs_a = _act(h_probs_l[t] / _TAU1)
        h_type_probs_a = _act(h_type_probs_l[t] / _TAU1)
        hid = _act(a_t_a[:, None] * (h_type_probs_a @ type_m.T)
                   + b_t_a[:, None] * (h_probs_a @ hidden_e_t))
        gate = 1.0 - _act(a_t_a + b_t_a)
        hiddens.append(hid + gate[:, None])

    a_t0_a = _act(a_t_l[0] / _TAU1)
    b_t0_a = _act(b_t_l[0] / _TAU1)
    h_x_probs_a = _act(h_x_probs / _TAU1)
    h_x_type_probs_a = _act(h_x_type_probs / _TAU1)
    hxv = _act(hx_acc)
    hxv = jnp.concatenate([hxv[:, half:-1], hxv[:, :half]], axis=-1)
    hidden_type_x = input_x @ (type_m @ h_x_type_probs_a.T)
    hidden_x0 = _act(a_t0_a[None, :] * hidden_type_x
                     + b_t0_a[None, :] * (hxv @ h_x_probs_a.T))
    gate_x = 1.0 - _act(_act(a_x / _TAU1) + _act(b_x / _TAU1))
    hidden_x0 = hidden_x0 + gate_x[None, :]

    s_b = (s0pre.reshape(Ln, batch, n_ent)
           * hiddens[0][:, None, :]
           * hidden_x0.T[:, :, None]).reshape(Ln * batch, n_ent)

    wgt = jnp.tanh(weight)[:, 0]
    for t in range(1, Tn):
        last = t == Tn - 1
        hid = hiddens[t] * wgt[:, None] if last else hiddens[t]
        hid8 = jnp.zeros((8, n_ent), jnp.float32).at[:Ln].set(hid)
        s_b = _pass_cd(t, last, s_b, e2t_b, t2et_b, sr_all, hid8)

    return s_b


# speedup vs baseline: 1.1367x; 1.1367x over previous
import functools

import jax
import jax.numpy as jnp
from jax.experimental import pallas as pl
from jax.experimental.pallas import tpu as pltpu

_TAU1 = 10.0
_TM = 640
_TN = 1280


def _act(v):
    return jnp.clip(v, 0.0, 1.0)


def _split_dot(a, b_bf16):
    a_hi = a.astype(jnp.bfloat16)
    a_lo = (a - a_hi.astype(jnp.float32)).astype(jnp.bfloat16)
    return (jnp.dot(a_hi, b_bf16, preferred_element_type=jnp.float32)
            + jnp.dot(a_lo, b_bf16, preferred_element_type=jnp.float32))


def _split_dot_nt(a, bt_bf16):
    nt = (((1,), (1,)), ((), ()))
    a_hi = a.astype(jnp.bfloat16)
    a_lo = (a - a_hi.astype(jnp.float32)).astype(jnp.bfloat16)
    return (jax.lax.dot_general(a_hi, bt_bf16, nt,
                                preferred_element_type=jnp.float32)
            + jax.lax.dot_general(a_lo, bt_bf16, nt,
                                  preferred_element_type=jnp.float32))


def _ab_kernel(e2t_ref, t2et_ref, t2rt_ref, x_ref, wall_ref,
               e2tb_ref, t2etb_ref, e2r_ref, s0_ref, hx_ref, sr_ref,
               srt_ref):
    nt = (((1,), (1,)), ((), ()))
    e2t_b = e2t_ref[...].astype(jnp.bfloat16)
    t2et_b = t2et_ref[...].astype(jnp.bfloat16)
    e2tb_ref[...] = e2t_b
    t2etb_ref[...] = t2et_b
    t2rt = t2rt_ref[...]
    t2rt_b = t2rt.astype(jnp.bfloat16)
    sr = jnp.dot(wall_ref[...], t2rt,
                 preferred_element_type=jnp.float32)
    sr_ref[...] = sr

    @pl.when(pl.program_id(0) == 0)
    def _():
        e2r_ref[...] = jnp.zeros_like(e2r_ref)
        s0_ref[...] = jnp.zeros_like(s0_ref)
        hx_ref[...] = jnp.zeros_like(hx_ref)
        srt_ref[...] = jnp.zeros_like(srt_ref)

    srt_ref[...] += _split_dot_nt(sr, t2et_b)
    e2r_ref[...] += jax.lax.dot_general(t2rt_b, t2et_b, nt,
                                        preferred_element_type=jnp.float32)
    xo = _split_dot(x_ref[...], e2t_b)
    hx_ref[...] += jax.lax.dot_general(xo.astype(jnp.bfloat16), t2rt_b, nt,
                                       preferred_element_type=jnp.float32)
    z = jnp.concatenate([xo * sr[l:l + 1, :] for l in range(3)],
                        axis=0)
    s0_ref[...] += _split_dot_nt(z, t2et_b)


def _pass_ab(e2triple, t2e_t, t2r_t, input_x, w_all):
    n_ent, n_tr = e2triple.shape
    n_rel = t2r_t.shape[0]
    batch = input_x.shape[0]
    grid = (n_tr // _TM,)
    return pl.pallas_call(
        _ab_kernel,
        grid=grid,
        in_specs=[
            pl.BlockSpec((n_ent, _TM), lambda i: (0, i)),
            pl.BlockSpec((n_ent, _TM), lambda i: (0, i)),
            pl.BlockSpec((n_rel, _TM), lambda i: (0, i)),
            pl.BlockSpec((batch, n_ent), lambda i: (0, 0)),
            pl.BlockSpec((16, n_rel), lambda i: (0, 0)),
        ],
        out_specs=[
            pl.BlockSpec((n_ent, _TM), lambda i: (0, i)),
            pl.BlockSpec((n_ent, _TM), lambda i: (0, i)),
            pl.BlockSpec((n_rel, n_ent), lambda i: (0, 0)),
            pl.BlockSpec((3 * batch, n_ent), lambda i: (0, 0)),
            pl.BlockSpec((batch, n_rel), lambda i: (0, 0)),
            pl.BlockSpec((16, _TM), lambda i: (0, i)),
            pl.BlockSpec((16, n_ent), lambda i: (0, 0)),
        ],
        out_shape=[
            jax.ShapeDtypeStruct((n_ent, n_tr), jnp.bfloat16),
            jax.ShapeDtypeStruct((n_ent, n_tr), jnp.bfloat16),
            jax.ShapeDtypeStruct((n_rel, n_ent), jnp.float32),
            jax.ShapeDtypeStruct((3 * batch, n_ent), jnp.float32),
            jax.ShapeDtypeStruct((batch, n_rel), jnp.float32),
            jax.ShapeDtypeStruct((16, n_tr), jnp.float32),
            jax.ShapeDtypeStruct((16, n_ent), jnp.float32),
        ],
    )(e2triple, t2e_t, t2r_t, input_x, w_all)


def _cd_kernel(t_idx, last, nsteps,
               smu_ref, sdel_ref, e2t_ref, t2et_ref, sr_ref, srt_ref,
               muy_ref, hid_ref, out_ref, acc_ref):
    i = pl.program_id(0)

    @pl.when(i == 0)
    def _():
        acc_ref[...] = jnp.zeros_like(acc_ref)

    nt = (((1,), (1,)), ((), ()))
    e2t = e2t_ref[...]
    t2et = t2et_ref[...]
    mu = smu_ref[:, 0:1]
    muy = muy_ref[:, 0:1]
    ones8 = jnp.ones((8, e2t.shape[0]), jnp.bfloat16)
    colsum = jnp.dot(ones8, e2t, preferred_element_type=jnp.float32)[0:1, :]
    y = (jnp.dot(sdel_ref[...], e2t, preferred_element_type=jnp.float32)
         + mu * colsum)
    sr3 = sr_ref[3 * t_idx:3 * t_idx + 3, :]
    dz = jnp.concatenate(
        [(y[32 * l:32 * (l + 1), :] - muy[32 * l:32 * (l + 1), :])
         * sr3[l:l + 1, :] for l in range(3)],
        axis=0).astype(jnp.bfloat16)
    acc_ref[...] += jax.lax.dot_general(dz, t2et, nt,
                                        preferred_element_type=jnp.float32)

    @pl.when(i == nsteps - 1)
    def _():
        acc = acc_ref[...]
        tot = acc + jnp.concatenate(
            [muy[32 * l:32 * (l + 1), :]
             * srt_ref[3 * t_idx + l:3 * t_idx + l + 1, :]
             for l in range(3)], axis=0)
        if last:
            out_ref[...] = (tot[0:32, :] * hid_ref[0:1, :]
                            + tot[32:64, :] * hid_ref[1:2, :]
                            + tot[64:96, :] * hid_ref[2:3, :])
        else:
            out_ref[...] = jnp.concatenate(
                [tot[32 * l:32 * (l + 1), :] * hid_ref[l:l + 1, :]
                 for l in range(3)], axis=0)


def _pass_cd(t_idx, last, s, e2t_b, t2et_b, sr_all, srt_all, hid):
    rows, n_ent = s.shape
    n_tr = e2t_b.shape[1]
    nsteps = n_tr // _TN
    mu = jnp.mean(s, axis=1, keepdims=True)
    s_mu = mu + jnp.zeros((rows, 128), jnp.float32)
    s_del = (s - mu).astype(jnp.bfloat16)
    muy = (0.5 * n_ent) * mu + jnp.zeros((rows, 128), jnp.float32)
    out_shape = (jax.ShapeDtypeStruct((32, n_ent), jnp.float32) if last
                 else jax.ShapeDtypeStruct((rows, n_ent), jnp.float32))
    out_rows = 32 if last else rows
    return pl.pallas_call(
        functools.partial(_cd_kernel, t_idx, last, nsteps),
        grid=(nsteps,),
        in_specs=[
            pl.BlockSpec((rows, 128), lambda i: (0, 0)),
            pl.BlockSpec((rows, n_ent), lambda i: (0, 0)),
            pl.BlockSpec((n_ent, _TN), lambda i: (0, i)),
            pl.BlockSpec((n_ent, _TN), lambda i: (0, i)),
            pl.BlockSpec((16, _TN), lambda i: (0, i)),
            pl.BlockSpec((16, n_ent), lambda i: (0, 0)),
            pl.BlockSpec((rows, 128), lambda i: (0, 0)),
            pl.BlockSpec((8, n_ent), lambda i: (0, 0)),
        ],
        out_specs=pl.BlockSpec((out_rows, n_ent), lambda i: (0, 0)),
        out_shape=out_shape,
        scratch_shapes=[pltpu.VMEM((rows, n_ent), jnp.float32)],
    )(s_mu, s_del, e2t_b, t2et_b, sr_all, srt_all, muy, hid)


def kernel(input_x, type, e2triple, triple2e, triple2r, w, weight, h, h_x,
           h_type, h_x_type, alpha, beta, alpha_x, beta_x, flag):
    type_m = type
    Tn, Ln, n = w.shape
    batch, n_ent = input_x.shape
    half = (n - 1) // 2
    flag_b = flag != 0

    w_probs_l, h_probs_l, h_type_probs_l, a_t_l, b_t_l = [], [], [], [], []
    for t in range(Tn):
        wp = jnp.where(flag_b, w[Tn - 1 - t], w[t])
        wp = jax.nn.softmax(wp, axis=-1)
        wp_flip = jnp.concatenate(
            [wp[:, half:-1], wp[:, :half], wp[:, -1:]], axis=-1)
        w_probs_l.append(jnp.where(flag_b, wp_flip, wp))
        if t == Tn - 1:
            h_probs_l.append(jnp.where(flag_b, h_x, h[t]))
            h_type_probs_l.append(jnp.where(flag_b, h_x_type, h_type[t]))
            a_t_l.append(jnp.where(flag_b, alpha_x, alpha[t]))
            b_t_l.append(jnp.where(flag_b, beta_x, beta[t]))
        else:
            h_probs_l.append(jnp.where(flag_b, h[Tn - 2 - t], h[t]))
            h_type_probs_l.append(jnp.where(flag_b, h_type[Tn - 2 - t],
                                            h_type[t]))
            a_t_l.append(jnp.where(flag_b, alpha[Tn - 2 - t], alpha[t]))
            b_t_l.append(jnp.where(flag_b, beta[Tn - 2 - t], beta[t]))
    h_x_probs = jnp.where(flag_b, h[-1], h_x)
    h_x_type_probs = jnp.where(flag_b, h_type[-1], h_x_type)
    a_x = jnp.where(flag_b, alpha[-1], alpha_x)
    b_x = jnp.where(flag_b, beta[-1], beta_x)

    w_stack = jnp.concatenate(w_probs_l, axis=0)
    w_all = jnp.zeros((16, n), jnp.float32).at[:3 * Tn].set(w_stack)

    e2t_b, t2et_b, e2r, s0pre, hx_acc, sr_all, srt_all = _pass_ab(
        e2triple, triple2e.T, triple2r.T, input_x, w_all)

    hidden_e_t = _act(e2r)[:-1, :]
    hiddens = []
    for t in range(Tn):
        a_t_a = _act(a_t_l[t] / _TAU1)
        b_t_a = _act(b_t_l[t] / _TAU1)
        h_probs_a = _act(h_probs_l[t] / _TAU1)
        h_type_probs_a = _act(h_type_probs_l[t] / _TAU1)
        hid = _act(a_t_a[:, None] * (h_type_probs_a @ type_m.T)
                   + b_t_a[:, None] * (h_probs_a @ hidden_e_t))
        gate = 1.0 - _act(a_t_a + b_t_a)
        hiddens.append(hid + gate[:, None])

    a_t0_a = _act(a_t_l[0] / _TAU1)
    b_t0_a = _act(b_t_l[0] / _TAU1)
    h_x_probs_a = _act(h_x_probs / _TAU1)
    h_x_type_probs_a = _act(h_x_type_probs / _TAU1)
    hxv = _act(hx_acc)
    hxv = jnp.concatenate([hxv[:, half:-1], hxv[:, :half]], axis=-1)
    hidden_type_x = input_x @ (type_m @ h_x_type_probs_a.T)
    hidden_x0 = _act(a_t0_a[None, :] * hidden_type_x
                     + b_t0_a[None, :] * (hxv @ h_x_probs_a.T))
    gate_x = 1.0 - _act(_act(a_x / _TAU1) + _act(b_x / _TAU1))
    hidden_x0 = hidden_x0 + gate_x[None, :]

    s_b = (s0pre.reshape(Ln, batch, n_ent)
           * hiddens[0][:, None, :]
           * hidden_x0.T[:, :, None]).reshape(Ln * batch, n_ent)

    wgt = jnp.tanh(weight)[:, 0]
    for t in range(1, Tn):
        last = t == Tn - 1
        hid = hiddens[t] * wgt[:, None] if last else hiddens[t]
        hid8 = jnp.zeros((8, n_ent), jnp.float32).at[:Ln].set(hid)
        s_b = _pass_cd(t, last, s_b, e2t_b, t2et_b, sr_all, srt_all, hid8)

    return s_b


# speedup vs baseline: 1.1779x; 1.0363x over previous
import functools

import jax
import jax.numpy as jnp
from jax.experimental import pallas as pl
from jax.experimental.pallas import tpu as pltpu

_TAU1 = 10.0
_TM = 640
_TN = 1280


def _act(v):
    return jnp.clip(v, 0.0, 1.0)


def _split_dot(a, b_bf16):
    a_hi = a.astype(jnp.bfloat16)
    a_lo = (a - a_hi.astype(jnp.float32)).astype(jnp.bfloat16)
    return (jnp.dot(a_hi, b_bf16, preferred_element_type=jnp.float32)
            + jnp.dot(a_lo, b_bf16, preferred_element_type=jnp.float32))


def _split_dot_nt(a, bt_bf16):
    nt = (((1,), (1,)), ((), ()))
    a_hi = a.astype(jnp.bfloat16)
    a_lo = (a - a_hi.astype(jnp.float32)).astype(jnp.bfloat16)
    return (jax.lax.dot_general(a_hi, bt_bf16, nt,
                                preferred_element_type=jnp.float32)
            + jax.lax.dot_general(a_lo, bt_bf16, nt,
                                  preferred_element_type=jnp.float32))


def _ab_kernel(e2t_ref, t2et_ref, t2rt_ref, x_ref, wall_ref, mux_ref,
               e2tb_ref, t2etb_ref, e2r_ref, s0_ref, hx_ref, sr_ref,
               srt_ref):
    nt = (((1,), (1,)), ((), ()))
    e2t_b = e2t_ref[...].astype(jnp.bfloat16)
    t2et_b = t2et_ref[...].astype(jnp.bfloat16)
    e2tb_ref[...] = e2t_b
    t2etb_ref[...] = t2et_b
    t2rt = t2rt_ref[...]
    t2rt_b = t2rt.astype(jnp.bfloat16)
    sr = jnp.dot(wall_ref[...], t2rt,
                 preferred_element_type=jnp.float32)
    sr_ref[...] = sr

    @pl.when(pl.program_id(0) == 0)
    def _():
        e2r_ref[...] = jnp.zeros_like(e2r_ref)
        s0_ref[...] = jnp.zeros_like(s0_ref)
        hx_ref[...] = jnp.zeros_like(hx_ref)
        srt_ref[...] = jnp.zeros_like(srt_ref)

    srt_ref[...] += _split_dot_nt(sr, t2et_b)
    e2r_ref[...] += jax.lax.dot_general(t2rt_b, t2et_b, nt,
                                        preferred_element_type=jnp.float32)
    xo = _split_dot(x_ref[...], e2t_b)
    hx_ref[...] += jax.lax.dot_general(xo.astype(jnp.bfloat16), t2rt_b, nt,
                                       preferred_element_type=jnp.float32)
    dxo = xo - mux_ref[:, 0:1]
    z = jnp.concatenate([dxo * sr[l:l + 1, :] for l in range(3)],
                        axis=0).astype(jnp.bfloat16)
    s0_ref[...] += jax.lax.dot_general(z, t2et_b, nt,
                                       preferred_element_type=jnp.float32)


def _pass_ab(e2triple, t2e_t, t2r_t, input_x, w_all, mux_col):
    n_ent, n_tr = e2triple.shape
    n_rel = t2r_t.shape[0]
    batch = input_x.shape[0]
    grid = (n_tr // _TM,)
    mux = mux_col + jnp.zeros((batch, 128), jnp.float32)
    return pl.pallas_call(
        _ab_kernel,
        grid=grid,
        in_specs=[
            pl.BlockSpec((n_ent, _TM), lambda i: (0, i)),
            pl.BlockSpec((n_ent, _TM), lambda i: (0, i)),
            pl.BlockSpec((n_rel, _TM), lambda i: (0, i)),
            pl.BlockSpec((batch, n_ent), lambda i: (0, 0)),
            pl.BlockSpec((16, n_rel), lambda i: (0, 0)),
            pl.BlockSpec((batch, 128), lambda i: (0, 0)),
        ],
        out_specs=[
            pl.BlockSpec((n_ent, _TM), lambda i: (0, i)),
            pl.BlockSpec((n_ent, _TM), lambda i: (0, i)),
            pl.BlockSpec((n_rel, n_ent), lambda i: (0, 0)),
            pl.BlockSpec((3 * batch, n_ent), lambda i: (0, 0)),
            pl.BlockSpec((batch, n_rel), lambda i: (0, 0)),
            pl.BlockSpec((16, _TM), lambda i: (0, i)),
            pl.BlockSpec((16, n_ent), lambda i: (0, 0)),
        ],
        out_shape=[
            jax.ShapeDtypeStruct((n_ent, n_tr), jnp.bfloat16),
            jax.ShapeDtypeStruct((n_ent, n_tr), jnp.bfloat16),
            jax.ShapeDtypeStruct((n_rel, n_ent), jnp.float32),
            jax.ShapeDtypeStruct((3 * batch, n_ent), jnp.float32),
            jax.ShapeDtypeStruct((batch, n_rel), jnp.float32),
            jax.ShapeDtypeStruct((16, n_tr), jnp.float32),
            jax.ShapeDtypeStruct((16, n_ent), jnp.float32),
        ],
    )(e2triple, t2e_t, t2r_t, input_x, w_all, mux)


def _cd_kernel(t_idx, last, nsteps,
               smu_ref, sdel_ref, e2t_ref, t2et_ref, sr_ref, srt_ref,
               muy_ref, hid_ref, out_ref, acc_ref):
    i = pl.program_id(0)

    @pl.when(i == 0)
    def _():
        acc_ref[...] = jnp.zeros_like(acc_ref)

    nt = (((1,), (1,)), ((), ()))
    e2t = e2t_ref[...]
    t2et = t2et_ref[...]
    mu = smu_ref[:, 0:1]
    muy = muy_ref[:, 0:1]
    ones8 = jnp.ones((8, e2t.shape[0]), jnp.bfloat16)
    colsum = jnp.dot(ones8, e2t, preferred_element_type=jnp.float32)[0:1, :]
    y = (jnp.dot(sdel_ref[...], e2t, preferred_element_type=jnp.float32)
         + mu * colsum)
    sr3 = sr_ref[3 * t_idx:3 * t_idx + 3, :]
    dz = jnp.concatenate(
        [(y[32 * l:32 * (l + 1), :] - muy[32 * l:32 * (l + 1), :])
         * sr3[l:l + 1, :] for l in range(3)],
        axis=0).astype(jnp.bfloat16)
    acc_ref[...] += jax.lax.dot_general(dz, t2et, nt,
                                        preferred_element_type=jnp.float32)

    @pl.when(i == nsteps - 1)
    def _():
        acc = acc_ref[...]
        tot = acc + jnp.concatenate(
            [muy[32 * l:32 * (l + 1), :]
             * srt_ref[3 * t_idx + l:3 * t_idx + l + 1, :]
             for l in range(3)], axis=0)
        if last:
            out_ref[...] = (tot[0:32, :] * hid_ref[0:1, :]
                            + tot[32:64, :] * hid_ref[1:2, :]
                            + tot[64:96, :] * hid_ref[2:3, :])
        else:
            out_ref[...] = jnp.concatenate(
                [tot[32 * l:32 * (l + 1), :] * hid_ref[l:l + 1, :]
                 for l in range(3)], axis=0)


def _pass_cd(t_idx, last, s, e2t_b, t2et_b, sr_all, srt_all, hid):
    rows, n_ent = s.shape
    n_tr = e2t_b.shape[1]
    nsteps = n_tr // _TN
    mu = jnp.mean(s, axis=1, keepdims=True)
    s_mu = mu + jnp.zeros((rows, 128), jnp.float32)
    s_del = (s - mu).astype(jnp.bfloat16)
    muy = (0.5 * n_ent) * mu + jnp.zeros((rows, 128), jnp.float32)
    out_shape = (jax.ShapeDtypeStruct((32, n_ent), jnp.float32) if last
                 else jax.ShapeDtypeStruct((rows, n_ent), jnp.float32))
    out_rows = 32 if last else rows
    return pl.pallas_call(
        functools.partial(_cd_kernel, t_idx, last, nsteps),
        grid=(nsteps,),
        in_specs=[
            pl.BlockSpec((rows, 128), lambda i: (0, 0)),
            pl.BlockSpec((rows, n_ent), lambda i: (0, 0)),
            pl.BlockSpec((n_ent, _TN), lambda i: (0, i)),
            pl.BlockSpec((n_ent, _TN), lambda i: (0, i)),
            pl.BlockSpec((16, _TN), lambda i: (0, i)),
            pl.BlockSpec((16, n_ent), lambda i: (0, 0)),
            pl.BlockSpec((rows, 128), lambda i: (0, 0)),
            pl.BlockSpec((8, n_ent), lambda i: (0, 0)),
        ],
        out_specs=pl.BlockSpec((out_rows, n_ent), lambda i: (0, 0)),
        out_shape=out_shape,
        scratch_shapes=[pltpu.VMEM((rows, n_ent), jnp.float32)],
    )(s_mu, s_del, e2t_b, t2et_b, sr_all, srt_all, muy, hid)


def kernel(input_x, type, e2triple, triple2e, triple2r, w, weight, h, h_x,
           h_type, h_x_type, alpha, beta, alpha_x, beta_x, flag):
    type_m = type
    Tn, Ln, n = w.shape
    batch, n_ent = input_x.shape
    half = (n - 1) // 2
    flag_b = flag != 0

    w_probs_l, h_probs_l, h_type_probs_l, a_t_l, b_t_l = [], [], [], [], []
    for t in range(Tn):
        wp = jnp.where(flag_b, w[Tn - 1 - t], w[t])
        wp = jax.nn.softmax(wp, axis=-1)
        wp_flip = jnp.concatenate(
            [wp[:, half:-1], wp[:, :half], wp[:, -1:]], axis=-1)
        w_probs_l.append(jnp.where(flag_b, wp_flip, wp))
        if t == Tn - 1:
            h_probs_l.append(jnp.where(flag_b, h_x, h[t]))
            h_type_probs_l.append(jnp.where(flag_b, h_x_type, h_type[t]))
            a_t_l.append(jnp.where(flag_b, alpha_x, alpha[t]))
            b_t_l.append(jnp.where(flag_b, beta_x, beta[t]))
        else:
            h_probs_l.append(jnp.where(flag_b, h[Tn - 2 - t], h[t]))
            h_type_probs_l.append(jnp.where(flag_b, h_type[Tn - 2 - t],
                                            h_type[t]))
            a_t_l.append(jnp.where(flag_b, alpha[Tn - 2 - t], alpha[t]))
            b_t_l.append(jnp.where(flag_b, beta[Tn - 2 - t], beta[t]))
    h_x_probs = jnp.where(flag_b, h[-1], h_x)
    h_x_type_probs = jnp.where(flag_b, h_type[-1], h_x_type)
    a_x = jnp.where(flag_b, alpha[-1], alpha_x)
    b_x = jnp.where(flag_b, beta[-1], beta_x)

    w_stack = jnp.concatenate(w_probs_l, axis=0)
    w_all = jnp.zeros((16, n), jnp.float32).at[:3 * Tn].set(w_stack)

    mux_col = 0.5 * jnp.sum(input_x, axis=1, keepdims=True)
    e2t_b, t2et_b, e2r, s0pre, hx_acc, sr_all, srt_all = _pass_ab(
        e2triple, triple2e.T, triple2r.T, input_x, w_all, mux_col)

    hidden_e_t = _act(e2r)[:-1, :]
    hiddens = []
    for t in range(Tn):
        a_t_a = _act(a_t_l[t] / _TAU1)
        b_t_a = _act(b_t_l[t] / _TAU1)
        h_probs_a = _act(h_probs_l[t] / _TAU1)
        h_type_probs_a = _act(h_type_probs_l[t] / _TAU1)
        hid = _act(a_t_a[:, None] * (h_type_probs_a @ type_m.T)
                   + b_t_a[:, None] * (h_probs_a @ hidden_e_t))
        gate = 1.0 - _act(a_t_a + b_t_a)
        hiddens.append(hid + gate[:, None])

    a_t0_a = _act(a_t_l[0] / _TAU1)
    b_t0_a = _act(b_t_l[0] / _TAU1)
    h_x_probs_a = _act(h_x_probs / _TAU1)
    h_x_type_probs_a = _act(h_x_type_probs / _TAU1)
    hxv = _act(hx_acc)
    hxv = jnp.concatenate([hxv[:, half:-1], hxv[:, :half]], axis=-1)
    hidden_type_x = input_x @ (type_m @ h_x_type_probs_a.T)
    hidden_x0 = _act(a_t0_a[None, :] * hidden_type_x
                     + b_t0_a[None, :] * (hxv @ h_x_probs_a.T))
    gate_x = 1.0 - _act(_act(a_x / _TAU1) + _act(b_x / _TAU1))
    hidden_x0 = hidden_x0 + gate_x[None, :]

    s0full = (s0pre.reshape(Ln, batch, n_ent)
              + mux_col[None, :, :] * srt_all[:Ln, None, :])
    s_b = (s0full
           * hiddens[0][:, None, :]
           * hidden_x0.T[:, :, None]).reshape(Ln * batch, n_ent)

    wgt = jnp.tanh(weight)[:, 0]
    for t in range(1, Tn):
        last = t == Tn - 1
        hid = hiddens[t] * wgt[:, None] if last else hiddens[t]
        hid8 = jnp.zeros((8, n_ent), jnp.float32).at[:Ln].set(hid)
        s_b = _pass_cd(t, last, s_b, e2t_b, t2et_b, sr_all, srt_all, hid8)

    return s_b
